# Initial kernel scaffold; baseline (speedup 1.0000x reference)
import functools

import jax
import jax.numpy as jnp
from jax import lax
from jax.experimental import pallas as pl
from jax.experimental.pallas import tpu as pltpu
from jax.experimental.pallas import tpu_sc as plsc

N_PART = 10000
N_EDGE = 320000
EMBED = 128
RBF_DIM = 16
OUT_EMBED = 256

NC = 2
NS = 16
NW = NC * NS
EPW = N_EDGE // NW
CHUNK = 80
NCHUNK = EPW // CHUNK
ROWS_PER_TILE = N_PART // NS
WB = 125

EDGE_BLK = 2000
NODE_BLK = 1000


def _edge_body(msg_ref, rbf_ref, wrbf_ref, out_ref):
    t = jnp.dot(rbf_ref[...], wrbf_ref[...], preferred_element_type=jnp.float32)
    out_ref[...] = msg_ref[...] * t


def _edge_product(messages, rbf, W_rbf):
    return pl.pallas_call(
        _edge_body,
        grid=(N_EDGE // EDGE_BLK,),
        in_specs=[
            pl.BlockSpec((EDGE_BLK, EMBED), lambda i: (i, 0)),
            pl.BlockSpec((EDGE_BLK, RBF_DIM), lambda i: (i, 0)),
            pl.BlockSpec((RBF_DIM, EMBED), lambda i: (0, 0)),
        ],
        out_specs=pl.BlockSpec((EDGE_BLK, EMBED), lambda i: (i, 0)),
        out_shape=jax.ShapeDtypeStruct((N_EDGE, EMBED), jnp.float32),
    )(messages, rbf, W_rbf)


_SC_MESH = plsc.VectorSubcoreMesh(core_axis_name="c", subcore_axis_name="s")


@functools.partial(
    pl.kernel,
    out_type=jax.ShapeDtypeStruct((NC, N_PART, EMBED), jnp.float32),
    mesh=_SC_MESH,
    scratch_types=[
        pltpu.VMEM_SHARED((N_PART, EMBED), jnp.float32),
        pltpu.VMEM((NCHUNK, CHUNK), jnp.int32),
        pltpu.VMEM((CHUNK, EMBED), jnp.float32),
        pltpu.VMEM((WB, EMBED), jnp.float32),
    ],
)
def _segment_sum_sc(m_hbm, idx_hbm, zeros_hbm, out_hbm, acc, idx_v, rows_v, tmp_v):
    c = lax.axis_index("c")
    s = lax.axis_index("s")
    w = c * NS + s

    pltpu.sync_copy(zeros_hbm, tmp_v)
    for k in range(ROWS_PER_TILE // WB):
        pltpu.sync_copy(tmp_v, acc.at[pl.ds(s * ROWS_PER_TILE + k * WB, WB)])
    plsc.subcore_barrier()

    pltpu.sync_copy(idx_hbm.at[w], idx_v)

    def body(j, carry):
        pltpu.sync_copy(m_hbm.at[pl.ds(w * EPW + j * CHUNK, CHUNK)], rows_v)
        pltpu.sync_copy(rows_v, acc.at[idx_v.at[j]], add=True)
        return carry

    lax.fori_loop(0, NCHUNK, body, 0)
    plsc.subcore_barrier()

    for k in range(ROWS_PER_TILE // WB):
        base = s * ROWS_PER_TILE + k * WB
        pltpu.sync_copy(acc.at[pl.ds(base, WB)], tmp_v)
        pltpu.sync_copy(tmp_v, out_hbm.at[c].at[pl.ds(base, WB)])


def _mlp_body(p0_ref, p1_ref, wup_ref, w1_ref, b1_ref, w2_ref, b2_ref,
              wf_ref, out_ref):
    summed = p0_ref[...] + p1_ref[...]
    u = jnp.dot(summed, wup_ref[...], preferred_element_type=jnp.float32)
    u = jnp.dot(u, w1_ref[...], preferred_element_type=jnp.float32) + b1_ref[...]
    u = u * jax.nn.sigmoid(u)
    u = jnp.dot(u, w2_ref[...], preferred_element_type=jnp.float32) + b2_ref[...]
    u = u * jax.nn.sigmoid(u)
    out_ref[...] = jnp.dot(u, wf_ref[...], preferred_element_type=jnp.float32)


def _node_mlp(p0, p1, W_up, W1, b1, W2, b2, W_final):
    return pl.pallas_call(
        _mlp_body,
        grid=(N_PART // NODE_BLK,),
        in_specs=[
            pl.BlockSpec((NODE_BLK, EMBED), lambda i: (i, 0)),
            pl.BlockSpec((NODE_BLK, EMBED), lambda i: (i, 0)),
            pl.BlockSpec((EMBED, OUT_EMBED), lambda i: (0, 0)),
            pl.BlockSpec((OUT_EMBED, OUT_EMBED), lambda i: (0, 0)),
            pl.BlockSpec((1, OUT_EMBED), lambda i: (0, 0)),
            pl.BlockSpec((OUT_EMBED, OUT_EMBED), lambda i: (0, 0)),
            pl.BlockSpec((1, OUT_EMBED), lambda i: (0, 0)),
            pl.BlockSpec((OUT_EMBED, 1), lambda i: (0, 0)),
        ],
        out_specs=pl.BlockSpec((NODE_BLK, 1), lambda i: (i, 0)),
        out_shape=jax.ShapeDtypeStruct((N_PART, 1), jnp.float32),
    )(p0, p1, W_up, W1, b1, W2, b2, W_final)


def kernel(messages, rbf, connectivity, W_rbf, W_up, W1, b1, W2, b2, W_final):
    idx = connectivity[0].astype(jnp.int32).reshape(NW, NCHUNK, CHUNK)
    m = _edge_product(messages, rbf, W_rbf)
    zeros = jnp.zeros((WB, EMBED), jnp.float32)
    partials = _segment_sum_sc(m, idx, zeros)
    return _node_mlp(partials[0], partials[1], W_up, W1,
                     b1.reshape(1, OUT_EMBED), W2, b2.reshape(1, OUT_EMBED),
                     W_final)


# speedup vs baseline: 2.2438x; 2.2438x over previous
import functools

import jax
import jax.numpy as jnp
from jax import lax
from jax.experimental import pallas as pl
from jax.experimental.pallas import tpu as pltpu
from jax.experimental.pallas import tpu_sc as plsc

N_PART = 10000
N_EDGE = 320000
EMBED = 128
RBF_DIM = 16
OUT_EMBED = 256

NC = 2
NS = 16
NW = NC * NS
EPW = N_EDGE // NW
CHUNK = 80
NCHUNK = EPW // CHUNK
ACC_ROWS = 10240
ROWS_PER_TILE = ACC_ROWS // NS
WB = 128

EDGE_BLK = 2000
NODE_BLK = 1000


def _edge_body(msg_ref, rbf_ref, wrbf_ref, out_ref):
    t = jnp.dot(rbf_ref[...], wrbf_ref[...], preferred_element_type=jnp.float32)
    out_ref[...] = msg_ref[...] * t


def _edge_product(messages, rbf, W_rbf):
    return pl.pallas_call(
        _edge_body,
        grid=(N_EDGE // EDGE_BLK,),
        in_specs=[
            pl.BlockSpec((EDGE_BLK, EMBED), lambda i: (i, 0)),
            pl.BlockSpec((EDGE_BLK, RBF_DIM), lambda i: (i, 0)),
            pl.BlockSpec((RBF_DIM, EMBED), lambda i: (0, 0)),
        ],
        out_specs=pl.BlockSpec((EDGE_BLK, EMBED), lambda i: (i, 0)),
        out_shape=jax.ShapeDtypeStruct((N_EDGE, EMBED), jnp.float32),
    )(messages, rbf, W_rbf)


_SC_MESH = plsc.VectorSubcoreMesh(core_axis_name="c", subcore_axis_name="s")


@functools.partial(
    pl.kernel,
    out_type=jax.ShapeDtypeStruct((NC, ACC_ROWS, EMBED), jnp.float32),
    mesh=_SC_MESH,
    scratch_types=[
        pltpu.VMEM_SHARED((ACC_ROWS, EMBED), jnp.float32),
        pltpu.VMEM((NCHUNK, CHUNK), jnp.int32),
        pltpu.VMEM((CHUNK, EMBED), jnp.float32),
        pltpu.VMEM((WB, EMBED), jnp.float32),
    ],
)
def _segment_sum_sc(m_hbm, idx_hbm, zeros_hbm, out_hbm, acc, idx_v, rows_v, tmp_v):
    c = lax.axis_index("c")
    s = lax.axis_index("s")
    w = c * NS + s

    pltpu.sync_copy(zeros_hbm, tmp_v)
    for k in range(ROWS_PER_TILE // WB):
        pltpu.sync_copy(tmp_v, acc.at[pl.ds(s * ROWS_PER_TILE + k * WB, WB)])
    plsc.subcore_barrier()

    pltpu.sync_copy(idx_hbm.at[w], idx_v)

    def body(j, carry):
        pltpu.sync_copy(m_hbm.at[pl.ds(w * EPW + j * CHUNK, CHUNK)], rows_v)
        pltpu.sync_copy(rows_v, acc.at[idx_v.at[j]], add=True)
        return carry

    lax.fori_loop(0, NCHUNK, body, 0)
    plsc.subcore_barrier()

    for k in range(ROWS_PER_TILE // WB):
        base = s * ROWS_PER_TILE + k * WB
        pltpu.sync_copy(acc.at[pl.ds(base, WB)], tmp_v)
        pltpu.sync_copy(tmp_v, out_hbm.at[c].at[pl.ds(base, WB)])


def _mlp_body(p0_ref, p1_ref, wup_ref, w1_ref, b1_ref, w2_ref, b2_ref,
              wf_ref, out_ref):
    summed = p0_ref[...] + p1_ref[...]
    u = jnp.dot(summed, wup_ref[...], preferred_element_type=jnp.float32)
    u = jnp.dot(u, w1_ref[...], preferred_element_type=jnp.float32) + b1_ref[...]
    u = u * jax.nn.sigmoid(u)
    u = jnp.dot(u, w2_ref[...], preferred_element_type=jnp.float32) + b2_ref[...]
    u = u * jax.nn.sigmoid(u)
    out_ref[...] = jnp.dot(u, wf_ref[...], preferred_element_type=jnp.float32)


def _node_mlp(p0, p1, W_up, W1, b1, W2, b2, W_final):
    return pl.pallas_call(
        _mlp_body,
        grid=(N_PART // NODE_BLK,),
        in_specs=[
            pl.BlockSpec((NODE_BLK, EMBED), lambda i: (i, 0)),
            pl.BlockSpec((NODE_BLK, EMBED), lambda i: (i, 0)),
            pl.BlockSpec((EMBED, OUT_EMBED), lambda i: (0, 0)),
            pl.BlockSpec((OUT_EMBED, OUT_EMBED), lambda i: (0, 0)),
            pl.BlockSpec((1, OUT_EMBED), lambda i: (0, 0)),
            pl.BlockSpec((OUT_EMBED, OUT_EMBED), lambda i: (0, 0)),
            pl.BlockSpec((1, OUT_EMBED), lambda i: (0, 0)),
            pl.BlockSpec((OUT_EMBED, 1), lambda i: (0, 0)),
        ],
        out_specs=pl.BlockSpec((NODE_BLK, 1), lambda i: (i, 0)),
        out_shape=jax.ShapeDtypeStruct((N_PART, 1), jnp.float32),
    )(p0, p1, W_up, W1, b1, W2, b2, W_final)


def kernel(messages, rbf, connectivity, W_rbf, W_up, W1, b1, W2, b2, W_final):
    idx = connectivity[0].astype(jnp.int32).reshape(NW, NCHUNK, CHUNK)
    m = _edge_product(messages, rbf, W_rbf)
    zeros = jnp.zeros((WB, EMBED), jnp.float32)
    partials = _segment_sum_sc(m, idx, zeros)
    return _node_mlp(partials[0, :N_PART], partials[1, :N_PART], W_up, W1,
                     b1.reshape(1, OUT_EMBED), W2, b2.reshape(1, OUT_EMBED),
                     W_final)


# speedup vs baseline: 2.9231x; 1.3027x over previous
import functools

import jax
import jax.numpy as jnp
from jax import lax
from jax.experimental import pallas as pl
from jax.experimental.pallas import tpu as pltpu
from jax.experimental.pallas import tpu_sc as plsc

N_PART = 10000
N_EDGE = 320000
EMBED = 128
RBF_DIM = 16
OUT_EMBED = 256

NC = 2
NS = 16
NW = NC * NS
EPW = N_EDGE // NW
CHUNK = 80
NCHUNK = EPW // CHUNK
ACC_ROWS = 10240
ROWS_PER_TILE = ACC_ROWS // NS
WB = 80

EDGE_BLK = 4000
NODE_BLK = 1000


def _edge_body(msg_ref, rbf_ref, wrbf_ref, out_ref):
    t = jnp.dot(rbf_ref[...], wrbf_ref[...], preferred_element_type=jnp.float32)
    out_ref[...] = msg_ref[...] * t


def _edge_product(messages, rbf, W_rbf):
    return pl.pallas_call(
        _edge_body,
        grid=(N_EDGE // EDGE_BLK,),
        in_specs=[
            pl.BlockSpec((EDGE_BLK, EMBED), lambda i: (i, 0)),
            pl.BlockSpec((EDGE_BLK, RBF_DIM), lambda i: (i, 0)),
            pl.BlockSpec((RBF_DIM, EMBED), lambda i: (0, 0)),
        ],
        out_specs=pl.BlockSpec((EDGE_BLK, EMBED), lambda i: (i, 0)),
        out_shape=jax.ShapeDtypeStruct((N_EDGE, EMBED), jnp.float32),
    )(messages, rbf, W_rbf)


_SC_MESH = plsc.VectorSubcoreMesh(core_axis_name="c", subcore_axis_name="s")


@functools.partial(
    pl.kernel,
    out_type=jax.ShapeDtypeStruct((NC, ACC_ROWS, EMBED), jnp.float32),
    mesh=_SC_MESH,
    scratch_types=[
        pltpu.VMEM_SHARED((ACC_ROWS, EMBED), jnp.float32),
        pltpu.VMEM((NCHUNK, CHUNK), jnp.int32),
        pltpu.VMEM((CHUNK, EMBED), jnp.float32),
        pltpu.VMEM((CHUNK, EMBED), jnp.float32),
        pltpu.VMEM((CHUNK, EMBED), jnp.float32),
        pltpu.SemaphoreType.DMA,
        pltpu.SemaphoreType.DMA,
        pltpu.SemaphoreType.DMA,
    ],
)
def _segment_sum_sc(m_hbm, idx_hbm, zeros_hbm, out_hbm, acc, idx_v, b0, b1, b2,
                    s0, s1, s2):
    c = lax.axis_index("c")
    s = lax.axis_index("s")
    w = c * NS + s
    ebase = w * EPW
    bufs = (b0, b1, b2)
    sems = (s0, s1, s2)

    pltpu.sync_copy(zeros_hbm, b0)
    for k in range(ROWS_PER_TILE // WB):
        pltpu.sync_copy(b0, acc.at[pl.ds(s * ROWS_PER_TILE + k * WB, WB)])
    plsc.subcore_barrier()

    pltpu.sync_copy(idx_hbm.at[w], idx_v)

    def _start(i, k):
        pltpu.async_copy(m_hbm.at[pl.ds(ebase + i * CHUNK, CHUNK)],
                         bufs[k], sems[k])

    def _wait(k):
        pltpu.make_async_copy(m_hbm.at[pl.ds(0, CHUNK)], bufs[k], sems[k]).wait()

    def _scatter(i, k):
        pltpu.sync_copy(bufs[k], acc.at[idx_v.at[i]], add=True)

    _start(0, 0)
    _start(1, 1)

    def body(g, carry):
        for k in range(3):
            i = 3 * g + k
            _wait(k)
            _scatter(i, k)
            _start(i + 2, (k + 2) % 3)
        return carry

    lax.fori_loop(0, NCHUNK // 3 - 1, body, 0)
    for i in range(NCHUNK - 5, NCHUNK):
        k = i % 3
        _wait(k)
        _scatter(i, k)
        if i + 2 < NCHUNK:
            _start(i + 2, (k + 2) % 3)
    plsc.subcore_barrier()

    for k in range(ROWS_PER_TILE // WB):
        base = s * ROWS_PER_TILE + k * WB
        pltpu.sync_copy(acc.at[pl.ds(base, WB)], b0)
        pltpu.sync_copy(b0, out_hbm.at[c].at[pl.ds(base, WB)])


def _mlp_body(p0_ref, p1_ref, wup_ref, w1_ref, b1_ref, w2_ref, b2_ref,
              wf_ref, out_ref):
    summed = p0_ref[...] + p1_ref[...]
    u = jnp.dot(summed, wup_ref[...], preferred_element_type=jnp.float32)
    u = jnp.dot(u, w1_ref[...], preferred_element_type=jnp.float32) + b1_ref[...]
    u = u * jax.nn.sigmoid(u)
    u = jnp.dot(u, w2_ref[...], preferred_element_type=jnp.float32) + b2_ref[...]
    u = u * jax.nn.sigmoid(u)
    out_ref[...] = jnp.dot(u, wf_ref[...], preferred_element_type=jnp.float32)


def _node_mlp(p0, p1, W_up, W1, b1, W2, b2, W_final):
    return pl.pallas_call(
        _mlp_body,
        grid=(N_PART // NODE_BLK,),
        in_specs=[
            pl.BlockSpec((NODE_BLK, EMBED), lambda i: (i, 0)),
            pl.BlockSpec((NODE_BLK, EMBED), lambda i: (i, 0)),
            pl.BlockSpec((EMBED, OUT_EMBED), lambda i: (0, 0)),
            pl.BlockSpec((OUT_EMBED, OUT_EMBED), lambda i: (0, 0)),
            pl.BlockSpec((1, OUT_EMBED), lambda i: (0, 0)),
            pl.BlockSpec((OUT_EMBED, OUT_EMBED), lambda i: (0, 0)),
            pl.BlockSpec((1, OUT_EMBED), lambda i: (0, 0)),
            pl.BlockSpec((OUT_EMBED, 1), lambda i: (0, 0)),
        ],
        out_specs=pl.BlockSpec((NODE_BLK, 1), lambda i: (i, 0)),
        out_shape=jax.ShapeDtypeStruct((N_PART, 1), jnp.float32),
    )(p0, p1, W_up, W1, b1, W2, b2, W_final)


def kernel(messages, rbf, connectivity, W_rbf, W_up, W1, b1, W2, b2, W_final):
    idx = connectivity[0].astype(jnp.int32).reshape(NW, NCHUNK, CHUNK)
    m = _edge_product(messages, rbf, W_rbf)
    zeros = jnp.zeros((WB, EMBED), jnp.float32)
    partials = _segment_sum_sc(m, idx, zeros)
    return _node_mlp(partials[0, :N_PART], partials[1, :N_PART], W_up, W1,
                     b1.reshape(1, OUT_EMBED), W2, b2.reshape(1, OUT_EMBED),
                     W_final)


# speedup vs baseline: 4.1481x; 1.4190x over previous
import functools

import jax
import jax.numpy as jnp
from jax import lax
from jax.experimental import pallas as pl
from jax.experimental.pallas import tpu as pltpu
from jax.experimental.pallas import tpu_sc as plsc

N_PART = 10000
N_EDGE = 320000
EMBED = 128
RBF_DIM = 16
OUT_EMBED = 256

NC = 2
NS = 16
NW = NC * NS
EPW = N_EDGE // NW
CHUNK = 80
NCHUNK = EPW // CHUNK
ACC_ROWS = 10240
ROWS_PER_TILE = ACC_ROWS // NS
WB = 80

EDGE_BLK = 3200
NODE_BLK = 1280


def _edge_body(msg_ref, rbft_ref, wrbf_ref, out_ref):
    t = lax.dot_general(rbft_ref[...], wrbf_ref[...],
                        (((0,), (0,)), ((), ())),
                        preferred_element_type=jnp.float32)
    out_ref[...] = msg_ref[...] * t


def _edge_product(messages, rbf_t, W_rbf):
    return pl.pallas_call(
        _edge_body,
        grid=(N_EDGE // EDGE_BLK,),
        in_specs=[
            pl.BlockSpec((EDGE_BLK, EMBED), lambda i: (i, 0)),
            pl.BlockSpec((RBF_DIM, EDGE_BLK), lambda i: (0, i)),
            pl.BlockSpec((RBF_DIM, EMBED), lambda i: (0, 0)),
        ],
        out_specs=pl.BlockSpec((EDGE_BLK, EMBED), lambda i: (i, 0)),
        out_shape=jax.ShapeDtypeStruct((N_EDGE, EMBED), jnp.float32),
    )(messages, rbf_t, W_rbf)


_SC_MESH = plsc.VectorSubcoreMesh(core_axis_name="c", subcore_axis_name="s")


@functools.partial(
    pl.kernel,
    out_type=jax.ShapeDtypeStruct((NC, ACC_ROWS, EMBED), jnp.float32),
    mesh=_SC_MESH,
    scratch_types=[
        pltpu.VMEM_SHARED((ACC_ROWS, EMBED), jnp.float32),
        pltpu.VMEM((NCHUNK, CHUNK), jnp.int32),
        pltpu.VMEM((CHUNK, EMBED), jnp.float32),
        pltpu.VMEM((CHUNK, EMBED), jnp.float32),
        pltpu.VMEM((CHUNK, EMBED), jnp.float32),
        pltpu.SemaphoreType.DMA,
        pltpu.SemaphoreType.DMA,
        pltpu.SemaphoreType.DMA,
        pltpu.SemaphoreType.DMA,
        pltpu.SemaphoreType.DMA,
        pltpu.SemaphoreType.DMA,
    ],
)
def _segment_sum_sc(m_hbm, idx_hbm, zeros_hbm, out_hbm, acc, idx_v, b0, b1, b2,
                    s0, s1, s2, t0, t1, t2):
    c = lax.axis_index("c")
    s = lax.axis_index("s")
    w = c * NS + s
    ebase = w * EPW
    bufs = (b0, b1, b2)
    sems = (s0, s1, s2)
    ssems = (t0, t1, t2)

    pltpu.sync_copy(zeros_hbm, b0)
    for k in range(ROWS_PER_TILE // WB):
        pltpu.sync_copy(b0, acc.at[pl.ds(s * ROWS_PER_TILE + k * WB, WB)])
    plsc.subcore_barrier()

    pltpu.sync_copy(idx_hbm.at[w], idx_v)

    def _start(i, k):
        pltpu.async_copy(m_hbm.at[pl.ds(ebase + i * CHUNK, CHUNK)],
                         bufs[k], sems[k])

    def _wait_load(k):
        pltpu.make_async_copy(m_hbm.at[pl.ds(0, CHUNK)], bufs[k], sems[k]).wait()

    def _scatter(i, k):
        pltpu.async_copy(bufs[k], acc.at[idx_v.at[i]], ssems[k], add=True)

    def _wait_scatter(k):
        pltpu.make_async_copy(bufs[k], acc.at[pl.ds(0, CHUNK)], ssems[k]).wait()

    _start(0, 0)
    _start(1, 1)
    _wait_load(0)
    _scatter(0, 0)
    _start(2, 2)
    _wait_load(1)
    _scatter(1, 1)
    _wait_scatter(0)
    _start(3, 0)
    _wait_load(2)
    _scatter(2, 2)
    _wait_scatter(1)
    _start(4, 1)

    def body(g, carry):
        for k in range(3):
            i = 3 * g + k
            _wait_load(k)
            _scatter(i, k)
            j = (k + 2) % 3
            _wait_scatter(j)
            _start(i + 2, j)
        return carry

    lax.fori_loop(1, NCHUNK // 3, body, 0)
    for i in range(NCHUNK - 2, NCHUNK):
        k = i % 3
        _wait_load(k)
        _scatter(i, k)
    for k in range(3):
        _wait_scatter(k)
    plsc.subcore_barrier()

    for k in range(ROWS_PER_TILE // WB):
        base = s * ROWS_PER_TILE + k * WB
        pltpu.sync_copy(acc.at[pl.ds(base, WB)], b0)
        pltpu.sync_copy(b0, out_hbm.at[c].at[pl.ds(base, WB)])


def _mlp_body(p0_ref, p1_ref, wup_ref, w1_ref, b1_ref, w2_ref, b2_ref,
              wf_ref, out_ref):
    summed = p0_ref[0] + p1_ref[0]
    u = jnp.dot(summed, wup_ref[...], preferred_element_type=jnp.float32)
    u = jnp.dot(u, w1_ref[...], preferred_element_type=jnp.float32) + b1_ref[...]
    u = u * jax.nn.sigmoid(u)
    u = jnp.dot(u, w2_ref[...], preferred_element_type=jnp.float32) + b2_ref[...]
    u = u * jax.nn.sigmoid(u)
    out_ref[...] = lax.dot_general(wf_ref[...], u, (((1,), (1,)), ((), ())),
                                   preferred_element_type=jnp.float32)


def _node_mlp(partials, W_up, W1, b1, W2, b2, W_final_row):
    return pl.pallas_call(
        _mlp_body,
        grid=(ACC_ROWS // NODE_BLK,),
        in_specs=[
            pl.BlockSpec((1, NODE_BLK, EMBED), lambda i: (0, i, 0)),
            pl.BlockSpec((1, NODE_BLK, EMBED), lambda i: (1, i, 0)),
            pl.BlockSpec((EMBED, OUT_EMBED), lambda i: (0, 0)),
            pl.BlockSpec((OUT_EMBED, OUT_EMBED), lambda i: (0, 0)),
            pl.BlockSpec((1, OUT_EMBED), lambda i: (0, 0)),
            pl.BlockSpec((OUT_EMBED, OUT_EMBED), lambda i: (0, 0)),
            pl.BlockSpec((1, OUT_EMBED), lambda i: (0, 0)),
            pl.BlockSpec((1, OUT_EMBED), lambda i: (0, 0)),
        ],
        out_specs=pl.BlockSpec((1, NODE_BLK), lambda i: (0, i)),
        out_shape=jax.ShapeDtypeStruct((1, ACC_ROWS), jnp.float32),
    )(partials, partials, W_up, W1, b1, W2, b2, W_final_row)


def kernel(messages, rbf, connectivity, W_rbf, W_up, W1, b1, W2, b2, W_final):
    idx = connectivity[0].astype(jnp.int32).reshape(NW, NCHUNK, CHUNK)
    m = _edge_product(messages, rbf.T, W_rbf)
    zeros = jnp.zeros((WB, EMBED), jnp.float32)
    partials = _segment_sum_sc(m, idx, zeros)
    out = _node_mlp(partials, W_up, W1, b1.reshape(1, OUT_EMBED), W2,
                    b2.reshape(1, OUT_EMBED), W_final.reshape(1, OUT_EMBED))
    return out[:, :N_PART].T
